# baseline (speedup 1.0000x reference)
import functools

import jax
import jax.numpy as jnp
from jax import lax
from jax.experimental import pallas as pl
from jax.experimental.pallas import tpu as pltpu
from jax.experimental.pallas import tpu_sc as plsc

T, D, E, DFF, DSH = 2048, 768, 64, 256, 512
TOPK = 2
NPAIR = T * TOPK
TB = 512
NTB = T // TB
BM = 64
NB = NPAIR // BM + E
NP = NB * BM

NWORK = 32
PAIR_W = NPAIR // NWORK
TOK_W = T // NWORK
CH2 = 16
NL = 16


def _router_shared_body(x_ref, gate_w_ref, Sg_ref, Su_ref, Sd_ref, sgw_ref,
                        idx_ref, w_ref, sh_ref):
    x = x_ref[...]
    logits = jnp.dot(x, gate_w_ref[...], preferred_element_type=jnp.float32)
    mx = jnp.max(logits, axis=-1, keepdims=True)
    p = jnp.exp(logits - mx)
    probs = p / jnp.sum(p, axis=-1, keepdims=True)
    iota_e = jax.lax.broadcasted_iota(jnp.int32, (TB, E), 1)
    m1 = jnp.max(probs, axis=-1, keepdims=True)
    i1 = jnp.min(jnp.where(probs == m1, iota_e, E), axis=-1)
    masked = jnp.where(iota_e == i1[:, None], -jnp.inf, probs)
    m2 = jnp.max(masked, axis=-1, keepdims=True)
    i2 = jnp.min(jnp.where(masked == m2, iota_e, E), axis=-1)
    denom = m1 + m2
    idx_ref[...] = jnp.concatenate([i1[:, None], i2[:, None]], axis=1)
    w_ref[...] = jnp.concatenate([m1 / denom, m2 / denom], axis=1)

    xb = x.astype(jnp.bfloat16)
    g = jnp.dot(xb, Sg_ref[...].astype(jnp.bfloat16),
                preferred_element_type=jnp.float32)
    u = jnp.dot(xb, Su_ref[...].astype(jnp.bfloat16),
                preferred_element_type=jnp.float32)
    h = (g * jax.nn.sigmoid(g) * u).astype(jnp.bfloat16)
    sh = jnp.dot(h, Sd_ref[...].astype(jnp.bfloat16),
                 preferred_element_type=jnp.float32)
    sgate = jax.nn.sigmoid(jnp.dot(x, sgw_ref[...],
                                   preferred_element_type=jnp.float32))
    sh_ref[...] = sgate * sh


def _router_shared(x, gate_w, Sg, Su, Sd, sgw):
    return pl.pallas_call(
        _router_shared_body,
        grid=(NTB,),
        in_specs=[
            pl.BlockSpec((TB, D), lambda t: (t, 0)),
            pl.BlockSpec((D, E), lambda t: (0, 0)),
            pl.BlockSpec((D, DSH), lambda t: (0, 0)),
            pl.BlockSpec((D, DSH), lambda t: (0, 0)),
            pl.BlockSpec((DSH, D), lambda t: (0, 0)),
            pl.BlockSpec((D, 1), lambda t: (0, 0)),
        ],
        out_specs=[
            pl.BlockSpec((TB, TOPK), lambda t: (t, 0)),
            pl.BlockSpec((TB, TOPK), lambda t: (t, 0)),
            pl.BlockSpec((TB, D), lambda t: (t, 0)),
        ],
        out_shape=[
            jax.ShapeDtypeStruct((T, TOPK), jnp.int32),
            jax.ShapeDtypeStruct((T, TOPK), jnp.float32),
            jax.ShapeDtypeStruct((T, D), jnp.float32),
        ],
        compiler_params=pltpu.CompilerParams(
            dimension_semantics=("arbitrary",)),
    )(x, gate_w, Sg, Su, Sd, sgw)


def _grouped_mlp_body(be_ref, x_ref, Wg_ref, Wu_ref, Wd_ref, w_ref, y_ref,
                      wg_s, wu_s, wd_s):
    b = pl.program_id(0)
    prev = be_ref[jnp.maximum(b - 1, 0)]

    @pl.when((b == 0) | (be_ref[b] != prev))
    def _recast():
        wg_s[...] = Wg_ref[0].astype(jnp.bfloat16)
        wu_s[...] = Wu_ref[0].astype(jnp.bfloat16)
        wd_s[...] = Wd_ref[0].astype(jnp.bfloat16)

    xb = x_ref[...].astype(jnp.bfloat16)
    g = jnp.dot(xb, wg_s[...], preferred_element_type=jnp.float32)
    u = jnp.dot(xb, wu_s[...], preferred_element_type=jnp.float32)
    h = (g * jax.nn.sigmoid(g) * u).astype(jnp.bfloat16)
    eo = jnp.dot(h, wd_s[...], preferred_element_type=jnp.float32)
    y_ref[...] = w_ref[...] * eo


def _grouped_mlp(block_expert, x_pad, Wg, Wu, Wd, w_pad):
    grid_spec = pltpu.PrefetchScalarGridSpec(
        num_scalar_prefetch=1,
        grid=(NB,),
        in_specs=[
            pl.BlockSpec((BM, D), lambda b, be: (b, 0)),
            pl.BlockSpec((1, D, DFF), lambda b, be: (be[b], 0, 0)),
            pl.BlockSpec((1, D, DFF), lambda b, be: (be[b], 0, 0)),
            pl.BlockSpec((1, DFF, D), lambda b, be: (be[b], 0, 0)),
            pl.BlockSpec((BM, 1), lambda b, be: (b, 0)),
        ],
        out_specs=pl.BlockSpec((BM, D), lambda b, be: (b, 0)),
        scratch_shapes=[
            pltpu.VMEM((D, DFF), jnp.bfloat16),
            pltpu.VMEM((D, DFF), jnp.bfloat16),
            pltpu.VMEM((DFF, D), jnp.bfloat16),
        ],
    )
    return pl.pallas_call(
        _grouped_mlp_body,
        grid_spec=grid_spec,
        out_shape=jax.ShapeDtypeStruct((NP, D), jnp.float32),
        compiler_params=pltpu.CompilerParams(
            dimension_semantics=("arbitrary",)),
    )(block_expert, x_pad, Wg, Wu, Wd, w_pad)


@functools.lru_cache(maxsize=1)
def _sc_kernels():
    mesh = plsc.VectorSubcoreMesh(core_axis_name="c", subcore_axis_name="s",
                                  num_cores=2)

    @functools.partial(
        pl.kernel,
        mesh=mesh,
        out_type=[
            jax.ShapeDtypeStruct((NP, D), jnp.float32),
            jax.ShapeDtypeStruct((NP,), jnp.float32),
        ],
        scratch_types=[
            pltpu.VMEM((PAIR_W,), jnp.int32),
            pltpu.VMEM((PAIR_W,), jnp.int32),
            pltpu.VMEM((PAIR_W,), jnp.int32),
            pltpu.VMEM((PAIR_W,), jnp.float32),
            pltpu.VMEM((PAIR_W, D), jnp.float32),
            pltpu.SemaphoreType.DMA,
        ],
    )
    def dispatch_x(order_hbm, pp_hbm, wflat_hbm, x_hbm, out_hbm, wpad_hbm,
                   ord_v, tok_v, pp_v, wv_v, rows_v, sem):
        wid = lax.axis_index("s") * 2 + lax.axis_index("c")
        base = wid * PAIR_W
        pltpu.sync_copy(order_hbm.at[pl.ds(base, PAIR_W)], ord_v)
        pltpu.sync_copy(pp_hbm.at[pl.ds(base, PAIR_W)], pp_v)
        for k in range(PAIR_W // NL):
            sl = pl.ds(k * NL, NL)
            tok_v[sl] = lax.shift_right_logical(ord_v[sl], 1)
        cp_w = pltpu.async_copy(wflat_hbm.at[ord_v], wv_v, sem)
        cp_x = pltpu.async_copy(x_hbm.at[tok_v], rows_v, sem)
        cp_w.wait()
        cp_x.wait()
        cp_w2 = pltpu.async_copy(wv_v, wpad_hbm.at[pp_v], sem)
        cp_x2 = pltpu.async_copy(rows_v, out_hbm.at[pp_v], sem)
        cp_w2.wait()
        cp_x2.wait()

    @functools.partial(
        pl.kernel,
        mesh=mesh,
        out_type=jax.ShapeDtypeStruct((T, D), jnp.float32),
        scratch_types=[
            pltpu.VMEM((TOK_W,), jnp.int32),
            pltpu.VMEM((TOK_W,), jnp.int32),
            [pltpu.VMEM((CH2, D), jnp.float32)] * 2,
            [pltpu.VMEM((CH2, D), jnp.float32)] * 2,
            [pltpu.VMEM((CH2, D), jnp.float32)] * 2,
            [pltpu.SemaphoreType.DMA] * 2,
        ],
    )
    def combine(pos0_hbm, pos1_hbm, y_hbm, sh_hbm, out_hbm,
                i0_v, i1_v, a_v, b_v, s_v, sem):
        wid = lax.axis_index("s") * 2 + lax.axis_index("c")
        base = wid * TOK_W
        nch = TOK_W // CH2
        pltpu.sync_copy(pos0_hbm.at[pl.ds(base, TOK_W)], i0_v)
        pltpu.sync_copy(pos1_hbm.at[pl.ds(base, TOK_W)], i1_v)

        def start(c, r):
            sl = pl.ds(c * CH2, CH2)
            pltpu.async_copy(y_hbm.at[i0_v.at[sl]], a_v[r], sem[r])
            pltpu.async_copy(y_hbm.at[i1_v.at[sl]], b_v[r], sem[r])
            pltpu.async_copy(sh_hbm.at[pl.ds(base + c * CH2, CH2)],
                             s_v[r], sem[r])

        def drain(r):
            pltpu.make_async_copy(y_hbm.at[i0_v.at[pl.ds(0, CH2)]],
                                  a_v[r], sem[r]).wait()
            pltpu.make_async_copy(y_hbm.at[i1_v.at[pl.ds(0, CH2)]],
                                  b_v[r], sem[r]).wait()
            pltpu.make_async_copy(sh_hbm.at[pl.ds(base, CH2)],
                                  s_v[r], sem[r]).wait()

        start(0, 0)
        for c in range(nch):
            r = c % 2
            if c + 1 < nch:
                start(c + 1, (c + 1) % 2)
            drain(r)

            def add_row(i, _):
                for k in range(D // NL):
                    sl = pl.ds(k * NL, NL)
                    a_v[r][i, sl] = (a_v[r][i, sl] + b_v[r][i, sl]
                                     + s_v[r][i, sl])
                return 0

            lax.fori_loop(0, CH2, add_row, 0)
            pltpu.sync_copy(a_v[r], out_hbm.at[pl.ds(base + c * CH2, CH2)])

    return dispatch_x, combine


def kernel(hidden_states, gate_w, Wg, Wu, Wd, Sg, Su, Sd, shared_gate_w):
    idx, w, sh = _router_shared(hidden_states, gate_w, Sg, Su, Sd,
                                shared_gate_w)

    flat_e = idx.reshape(-1)
    e_sorted, order = lax.sort(
        (flat_e, jnp.arange(NPAIR, dtype=jnp.int32)),
        dimension=0, is_stable=True, num_keys=1)
    ee = jnp.arange(E, dtype=jnp.int32)
    grp_start = jnp.searchsorted(e_sorted, ee, side="left").astype(jnp.int32)
    counts = (jnp.concatenate([grp_start[1:],
                               jnp.array([NPAIR], jnp.int32)]) - grp_start)
    nblk = (counts + BM - 1) // BM
    blk_end = jnp.cumsum(nblk)
    pad_off = ((blk_end - nblk) * BM).astype(jnp.int32)
    block_expert = jnp.minimum(
        jnp.searchsorted(blk_end, jnp.arange(NB, dtype=jnp.int32),
                         side="right").astype(jnp.int32), E - 1)
    delta_tbl = pad_off - grp_start
    delta = jnp.sum(jnp.where(e_sorted[:, None] == ee[None, :],
                              delta_tbl[None, :], 0), axis=1, dtype=jnp.int32)
    pp = jnp.arange(NPAIR, dtype=jnp.int32) + delta
    posf = jnp.zeros((NPAIR,), jnp.int32).at[order].set(pp)
    pos = posf.reshape(T, TOPK)

    dispatch_x, combine = _sc_kernels()
    x_pad, w_pad = dispatch_x(order, pp, w.reshape(-1), hidden_states)
    y = _grouped_mlp(block_expert, x_pad, Wg, Wu, Wd, w_pad[:, None])
    return combine(pos[:, 0], pos[:, 1], y, sh)


# speedup vs baseline: 1.1463x; 1.1463x over previous
import functools

import jax
import jax.numpy as jnp
from jax import lax
from jax.experimental import pallas as pl
from jax.experimental.pallas import tpu as pltpu
from jax.experimental.pallas import tpu_sc as plsc

T, D, E, DFF, DSH = 2048, 768, 64, 256, 512
TOPK = 2
NPAIR = T * TOPK
TB = 512
NTB = T // TB
BM = 128
NB = NPAIR // BM + E
NP = NB * BM

NWORK = 32
PAIR_W = NPAIR // NWORK
TOK_W = T // NWORK
CH2 = 16
NL = 16


def _router_shared_body(x_ref, gate_w_ref, Sg_ref, Su_ref, Sd_ref, sgw_ref,
                        idx_ref, w_ref, sh_ref):
    x = x_ref[...]
    logits = jnp.dot(x, gate_w_ref[...], preferred_element_type=jnp.float32)
    mx = jnp.max(logits, axis=-1, keepdims=True)
    p = jnp.exp(logits - mx)
    probs = p / jnp.sum(p, axis=-1, keepdims=True)
    iota_e = jax.lax.broadcasted_iota(jnp.int32, (TB, E), 1)
    m1 = jnp.max(probs, axis=-1, keepdims=True)
    i1 = jnp.min(jnp.where(probs == m1, iota_e, E), axis=-1)
    masked = jnp.where(iota_e == i1[:, None], -jnp.inf, probs)
    m2 = jnp.max(masked, axis=-1, keepdims=True)
    i2 = jnp.min(jnp.where(masked == m2, iota_e, E), axis=-1)
    denom = m1 + m2
    idx_ref[...] = jnp.concatenate([i1[:, None], i2[:, None]], axis=1)
    w_ref[...] = jnp.concatenate([m1 / denom, m2 / denom], axis=1)

    xb = x.astype(jnp.bfloat16)
    g = jnp.dot(xb, Sg_ref[...].astype(jnp.bfloat16),
                preferred_element_type=jnp.float32)
    u = jnp.dot(xb, Su_ref[...].astype(jnp.bfloat16),
                preferred_element_type=jnp.float32)
    h = (g * jax.nn.sigmoid(g) * u).astype(jnp.bfloat16)
    sh = jnp.dot(h, Sd_ref[...].astype(jnp.bfloat16),
                 preferred_element_type=jnp.float32)
    sgate = jax.nn.sigmoid(jnp.dot(x, sgw_ref[...],
                                   preferred_element_type=jnp.float32))
    sh_ref[...] = sgate * sh


def _router_shared(x, gate_w, Sg, Su, Sd, sgw):
    return pl.pallas_call(
        _router_shared_body,
        grid=(NTB,),
        in_specs=[
            pl.BlockSpec((TB, D), lambda t: (t, 0)),
            pl.BlockSpec((D, E), lambda t: (0, 0)),
            pl.BlockSpec((D, DSH), lambda t: (0, 0)),
            pl.BlockSpec((D, DSH), lambda t: (0, 0)),
            pl.BlockSpec((DSH, D), lambda t: (0, 0)),
            pl.BlockSpec((D, 1), lambda t: (0, 0)),
        ],
        out_specs=[
            pl.BlockSpec((TB, TOPK), lambda t: (t, 0)),
            pl.BlockSpec((TB, TOPK), lambda t: (t, 0)),
            pl.BlockSpec((TB, D), lambda t: (t, 0)),
        ],
        out_shape=[
            jax.ShapeDtypeStruct((T, TOPK), jnp.int32),
            jax.ShapeDtypeStruct((T, TOPK), jnp.float32),
            jax.ShapeDtypeStruct((T, D), jnp.float32),
        ],
        compiler_params=pltpu.CompilerParams(
            dimension_semantics=("arbitrary",)),
    )(x, gate_w, Sg, Su, Sd, sgw)


def _grouped_mlp_body(be_ref, x_ref, Wg_ref, Wu_ref, Wd_ref, w_ref, y_ref,
                      wg_s, wu_s, wd_s):
    b = pl.program_id(0)
    prev = be_ref[jnp.maximum(b - 1, 0)]

    @pl.when((b == 0) | (be_ref[b] != prev))
    def _recast():
        wg_s[...] = Wg_ref[0].astype(jnp.bfloat16)
        wu_s[...] = Wu_ref[0].astype(jnp.bfloat16)
        wd_s[...] = Wd_ref[0].astype(jnp.bfloat16)

    xb = x_ref[...].astype(jnp.bfloat16)
    g = jnp.dot(xb, wg_s[...], preferred_element_type=jnp.float32)
    u = jnp.dot(xb, wu_s[...], preferred_element_type=jnp.float32)
    h = (g * jax.nn.sigmoid(g) * u).astype(jnp.bfloat16)
    eo = jnp.dot(h, wd_s[...], preferred_element_type=jnp.float32)
    y_ref[...] = w_ref[...] * eo


def _grouped_mlp(block_expert, x_pad, Wg, Wu, Wd, w_pad):
    grid_spec = pltpu.PrefetchScalarGridSpec(
        num_scalar_prefetch=1,
        grid=(NB,),
        in_specs=[
            pl.BlockSpec((BM, D), lambda b, be: (b, 0)),
            pl.BlockSpec((1, D, DFF), lambda b, be: (be[b], 0, 0)),
            pl.BlockSpec((1, D, DFF), lambda b, be: (be[b], 0, 0)),
            pl.BlockSpec((1, DFF, D), lambda b, be: (be[b], 0, 0)),
            pl.BlockSpec((BM, 1), lambda b, be: (b, 0)),
        ],
        out_specs=pl.BlockSpec((BM, D), lambda b, be: (b, 0)),
        scratch_shapes=[
            pltpu.VMEM((D, DFF), jnp.bfloat16),
            pltpu.VMEM((D, DFF), jnp.bfloat16),
            pltpu.VMEM((DFF, D), jnp.bfloat16),
        ],
    )
    return pl.pallas_call(
        _grouped_mlp_body,
        grid_spec=grid_spec,
        out_shape=jax.ShapeDtypeStruct((NP, D), jnp.float32),
        compiler_params=pltpu.CompilerParams(
            dimension_semantics=("arbitrary",)),
    )(block_expert, x_pad, Wg, Wu, Wd, w_pad)


@functools.lru_cache(maxsize=1)
def _sc_kernels():
    mesh = plsc.VectorSubcoreMesh(core_axis_name="c", subcore_axis_name="s",
                                  num_cores=2)

    @functools.partial(
        pl.kernel,
        mesh=mesh,
        out_type=[
            jax.ShapeDtypeStruct((NP, D), jnp.float32),
            jax.ShapeDtypeStruct((NP,), jnp.float32),
        ],
        scratch_types=[
            pltpu.VMEM((PAIR_W,), jnp.int32),
            pltpu.VMEM((PAIR_W,), jnp.int32),
            pltpu.VMEM((PAIR_W,), jnp.int32),
            pltpu.VMEM((PAIR_W,), jnp.float32),
            pltpu.VMEM((PAIR_W, D), jnp.float32),
            pltpu.SemaphoreType.DMA,
        ],
    )
    def dispatch_x(order_hbm, pp_hbm, wflat_hbm, x_hbm, out_hbm, wpad_hbm,
                   ord_v, tok_v, pp_v, wv_v, rows_v, sem):
        wid = lax.axis_index("s") * 2 + lax.axis_index("c")
        base = wid * PAIR_W
        pltpu.sync_copy(order_hbm.at[pl.ds(base, PAIR_W)], ord_v)
        pltpu.sync_copy(pp_hbm.at[pl.ds(base, PAIR_W)], pp_v)
        for k in range(PAIR_W // NL):
            sl = pl.ds(k * NL, NL)
            tok_v[sl] = lax.shift_right_logical(ord_v[sl], 1)
        cp_w = pltpu.async_copy(wflat_hbm.at[ord_v], wv_v, sem)
        cp_x = pltpu.async_copy(x_hbm.at[tok_v], rows_v, sem)
        cp_w.wait()
        cp_x.wait()
        cp_w2 = pltpu.async_copy(wv_v, wpad_hbm.at[pp_v], sem)
        cp_x2 = pltpu.async_copy(rows_v, out_hbm.at[pp_v], sem)
        cp_w2.wait()
        cp_x2.wait()

    @functools.partial(
        pl.kernel,
        mesh=mesh,
        out_type=jax.ShapeDtypeStruct((T, D), jnp.float32),
        scratch_types=[
            pltpu.VMEM((TOK_W,), jnp.int32),
            pltpu.VMEM((TOK_W,), jnp.int32),
            [pltpu.VMEM((CH2, D), jnp.float32)] * 2,
            [pltpu.VMEM((CH2, D), jnp.float32)] * 2,
            [pltpu.VMEM((CH2, D), jnp.float32)] * 2,
            [pltpu.SemaphoreType.DMA] * 2,
        ],
    )
    def combine(pos0_hbm, pos1_hbm, y_hbm, sh_hbm, out_hbm,
                i0_v, i1_v, a_v, b_v, s_v, sem):
        wid = lax.axis_index("s") * 2 + lax.axis_index("c")
        base = wid * TOK_W
        nch = TOK_W // CH2
        pltpu.sync_copy(pos0_hbm.at[pl.ds(base, TOK_W)], i0_v)
        pltpu.sync_copy(pos1_hbm.at[pl.ds(base, TOK_W)], i1_v)

        def start(c, r):
            sl = pl.ds(c * CH2, CH2)
            pltpu.async_copy(y_hbm.at[i0_v.at[sl]], a_v[r], sem[r])
            pltpu.async_copy(y_hbm.at[i1_v.at[sl]], b_v[r], sem[r])
            pltpu.async_copy(sh_hbm.at[pl.ds(base + c * CH2, CH2)],
                             s_v[r], sem[r])

        def drain(r):
            pltpu.make_async_copy(y_hbm.at[i0_v.at[pl.ds(0, CH2)]],
                                  a_v[r], sem[r]).wait()
            pltpu.make_async_copy(y_hbm.at[i1_v.at[pl.ds(0, CH2)]],
                                  b_v[r], sem[r]).wait()
            pltpu.make_async_copy(sh_hbm.at[pl.ds(base, CH2)],
                                  s_v[r], sem[r]).wait()

        start(0, 0)
        for c in range(nch):
            r = c % 2
            if c + 1 < nch:
                start(c + 1, (c + 1) % 2)
            drain(r)

            def add_row(i, _):
                for k in range(D // NL):
                    sl = pl.ds(k * NL, NL)
                    a_v[r][i, sl] = (a_v[r][i, sl] + b_v[r][i, sl]
                                     + s_v[r][i, sl])
                return 0

            lax.fori_loop(0, CH2, add_row, 0)
            pltpu.sync_copy(a_v[r], out_hbm.at[pl.ds(base + c * CH2, CH2)])

    return dispatch_x, combine


def kernel(hidden_states, gate_w, Wg, Wu, Wd, Sg, Su, Sd, shared_gate_w):
    idx, w, sh = _router_shared(hidden_states, gate_w, Sg, Su, Sd,
                                shared_gate_w)

    flat_e = idx.reshape(-1)
    e_sorted, order = lax.sort(
        (flat_e, jnp.arange(NPAIR, dtype=jnp.int32)),
        dimension=0, is_stable=True, num_keys=1)
    ee = jnp.arange(E, dtype=jnp.int32)
    grp_start = jnp.searchsorted(e_sorted, ee, side="left").astype(jnp.int32)
    counts = (jnp.concatenate([grp_start[1:],
                               jnp.array([NPAIR], jnp.int32)]) - grp_start)
    nblk = (counts + BM - 1) // BM
    blk_end = jnp.cumsum(nblk)
    pad_off = ((blk_end - nblk) * BM).astype(jnp.int32)
    block_expert = jnp.minimum(
        jnp.searchsorted(blk_end, jnp.arange(NB, dtype=jnp.int32),
                         side="right").astype(jnp.int32), E - 1)
    delta_tbl = pad_off - grp_start
    delta = jnp.sum(jnp.where(e_sorted[:, None] == ee[None, :],
                              delta_tbl[None, :], 0), axis=1, dtype=jnp.int32)
    pp = jnp.arange(NPAIR, dtype=jnp.int32) + delta
    posf = jnp.zeros((NPAIR,), jnp.int32).at[order].set(pp)
    pos = posf.reshape(T, TOPK)

    dispatch_x, combine = _sc_kernels()
    x_pad, w_pad = dispatch_x(order, pp, w.reshape(-1), hidden_states)
    y = _grouped_mlp(block_expert, x_pad, Wg, Wu, Wd, w_pad[:, None])
    return combine(pos[:, 0], pos[:, 1], y, sh)


# speedup vs baseline: 1.1722x; 1.0227x over previous
import functools

import jax
import jax.numpy as jnp
from jax import lax
from jax.experimental import pallas as pl
from jax.experimental.pallas import tpu as pltpu
from jax.experimental.pallas import tpu_sc as plsc

T, D, E, DFF, DSH = 2048, 768, 64, 256, 512
TOPK = 2
NPAIR = T * TOPK
TB = 512
NTB = T // TB
BM = 128
NB = NPAIR // BM + E
NP = NB * BM

NWORK = 32
PAIR_W = NPAIR // NWORK
TOK_W = T // NWORK
CH2 = 16
NL = 16


def _router_body(x_ref, gate_w_ref, idx_ref, w_ref):
    x = x_ref[...]
    logits = jnp.dot(x, gate_w_ref[...], preferred_element_type=jnp.float32)
    mx = jnp.max(logits, axis=-1, keepdims=True)
    p = jnp.exp(logits - mx)
    probs = p / jnp.sum(p, axis=-1, keepdims=True)
    iota_e = jax.lax.broadcasted_iota(jnp.int32, (T, E), 1)
    m1 = jnp.max(probs, axis=-1, keepdims=True)
    i1 = jnp.min(jnp.where(probs == m1, iota_e, E), axis=-1)
    masked = jnp.where(iota_e == i1[:, None], -jnp.inf, probs)
    m2 = jnp.max(masked, axis=-1, keepdims=True)
    i2 = jnp.min(jnp.where(masked == m2, iota_e, E), axis=-1)
    denom = m1 + m2
    idx_ref[...] = jnp.concatenate([i1[:, None], i2[:, None]], axis=1)
    w_ref[...] = jnp.concatenate([m1 / denom, m2 / denom], axis=1)


def _router(x, gate_w):
    return pl.pallas_call(
        _router_body,
        out_shape=[
            jax.ShapeDtypeStruct((T, TOPK), jnp.int32),
            jax.ShapeDtypeStruct((T, TOPK), jnp.float32),
        ],
    )(x, gate_w)


def _shared_body(x_ref, Sg_ref, Su_ref, Sd_ref, sgw_ref, sh_ref):
    x = x_ref[...]
    xb = x.astype(jnp.bfloat16)
    g = jnp.dot(xb, Sg_ref[...].astype(jnp.bfloat16),
                preferred_element_type=jnp.float32)
    u = jnp.dot(xb, Su_ref[...].astype(jnp.bfloat16),
                preferred_element_type=jnp.float32)
    h = (g * jax.nn.sigmoid(g) * u).astype(jnp.bfloat16)
    sh = jnp.dot(h, Sd_ref[...].astype(jnp.bfloat16),
                 preferred_element_type=jnp.float32)
    sgate = jax.nn.sigmoid(jnp.dot(x, sgw_ref[...],
                                   preferred_element_type=jnp.float32))
    sh_ref[...] = sgate * sh


def _shared(x, Sg, Su, Sd, sgw):
    return pl.pallas_call(
        _shared_body,
        grid=(NTB,),
        in_specs=[
            pl.BlockSpec((TB, D), lambda t: (t, 0)),
            pl.BlockSpec((D, DSH), lambda t: (0, 0)),
            pl.BlockSpec((D, DSH), lambda t: (0, 0)),
            pl.BlockSpec((DSH, D), lambda t: (0, 0)),
            pl.BlockSpec((D, 1), lambda t: (0, 0)),
        ],
        out_specs=pl.BlockSpec((TB, D), lambda t: (t, 0)),
        out_shape=jax.ShapeDtypeStruct((T, D), jnp.float32),
        compiler_params=pltpu.CompilerParams(
            dimension_semantics=("arbitrary",)),
    )(x, Sg, Su, Sd, sgw)


def _grouped_mlp_body(be_ref, x_ref, Wg_ref, Wu_ref, Wd_ref, w_ref, y_ref,
                      wg_s, wu_s, wd_s):
    b = pl.program_id(0)
    prev = be_ref[jnp.maximum(b - 1, 0)]

    @pl.when((b == 0) | (be_ref[b] != prev))
    def _recast():
        wg_s[...] = Wg_ref[0].astype(jnp.bfloat16)
        wu_s[...] = Wu_ref[0].astype(jnp.bfloat16)
        wd_s[...] = Wd_ref[0].astype(jnp.bfloat16)

    xb = x_ref[...].astype(jnp.bfloat16)
    g = jnp.dot(xb, wg_s[...], preferred_element_type=jnp.float32)
    u = jnp.dot(xb, wu_s[...], preferred_element_type=jnp.float32)
    h = (g * jax.nn.sigmoid(g) * u).astype(jnp.bfloat16)
    eo = jnp.dot(h, wd_s[...], preferred_element_type=jnp.float32)
    y_ref[...] = w_ref[...] * eo


def _grouped_mlp(block_expert, x_pad, Wg, Wu, Wd, w_pad):
    grid_spec = pltpu.PrefetchScalarGridSpec(
        num_scalar_prefetch=1,
        grid=(NB,),
        in_specs=[
            pl.BlockSpec((BM, D), lambda b, be: (b, 0)),
            pl.BlockSpec((1, D, DFF), lambda b, be: (be[b], 0, 0)),
            pl.BlockSpec((1, D, DFF), lambda b, be: (be[b], 0, 0)),
            pl.BlockSpec((1, DFF, D), lambda b, be: (be[b], 0, 0)),
            pl.BlockSpec((BM, 1), lambda b, be: (b, 0)),
        ],
        out_specs=pl.BlockSpec((BM, D), lambda b, be: (b, 0)),
        scratch_shapes=[
            pltpu.VMEM((D, DFF), jnp.bfloat16),
            pltpu.VMEM((D, DFF), jnp.bfloat16),
            pltpu.VMEM((DFF, D), jnp.bfloat16),
        ],
    )
    return pl.pallas_call(
        _grouped_mlp_body,
        grid_spec=grid_spec,
        out_shape=jax.ShapeDtypeStruct((NP, D), jnp.float32),
        compiler_params=pltpu.CompilerParams(
            dimension_semantics=("arbitrary",)),
    )(block_expert, x_pad, Wg, Wu, Wd, w_pad)


@functools.lru_cache(maxsize=1)
def _sc_kernels():
    mesh = plsc.VectorSubcoreMesh(core_axis_name="c", subcore_axis_name="s",
                                  num_cores=2)

    @functools.partial(
        pl.kernel,
        mesh=mesh,
        out_type=[
            jax.ShapeDtypeStruct((NP, D), jnp.float32),
            jax.ShapeDtypeStruct((NP,), jnp.float32),
        ],
        scratch_types=[
            pltpu.VMEM((PAIR_W,), jnp.int32),
            pltpu.VMEM((PAIR_W,), jnp.int32),
            pltpu.VMEM((PAIR_W,), jnp.int32),
            pltpu.VMEM((PAIR_W,), jnp.float32),
            pltpu.VMEM((PAIR_W, D), jnp.float32),
            pltpu.SemaphoreType.DMA,
        ],
    )
    def dispatch_x(order_hbm, pp_hbm, wflat_hbm, x_hbm, out_hbm, wpad_hbm,
                   ord_v, tok_v, pp_v, wv_v, rows_v, sem):
        wid = lax.axis_index("s") * 2 + lax.axis_index("c")
        base = wid * PAIR_W
        pltpu.sync_copy(order_hbm.at[pl.ds(base, PAIR_W)], ord_v)
        pltpu.sync_copy(pp_hbm.at[pl.ds(base, PAIR_W)], pp_v)
        for k in range(PAIR_W // NL):
            sl = pl.ds(k * NL, NL)
            tok_v[sl] = lax.shift_right_logical(ord_v[sl], 1)
        cp_w = pltpu.async_copy(wflat_hbm.at[ord_v], wv_v, sem)
        cp_x = pltpu.async_copy(x_hbm.at[tok_v], rows_v, sem)
        cp_w.wait()
        cp_x.wait()
        cp_w2 = pltpu.async_copy(wv_v, wpad_hbm.at[pp_v], sem)
        cp_x2 = pltpu.async_copy(rows_v, out_hbm.at[pp_v], sem)
        cp_w2.wait()
        cp_x2.wait()

    @functools.partial(
        pl.kernel,
        mesh=mesh,
        out_type=jax.ShapeDtypeStruct((T, D), jnp.float32),
        scratch_types=[
            pltpu.VMEM((TOK_W,), jnp.int32),
            pltpu.VMEM((TOK_W,), jnp.int32),
            [pltpu.VMEM((CH2, D), jnp.float32)] * 2,
            [pltpu.VMEM((CH2, D), jnp.float32)] * 2,
            [pltpu.VMEM((CH2, D), jnp.float32)] * 2,
            [pltpu.SemaphoreType.DMA] * 2,
        ],
    )
    def combine(pos0_hbm, pos1_hbm, y_hbm, sh_hbm, out_hbm,
                i0_v, i1_v, a_v, b_v, s_v, sem):
        wid = lax.axis_index("s") * 2 + lax.axis_index("c")
        base = wid * TOK_W
        nch = TOK_W // CH2
        pltpu.sync_copy(pos0_hbm.at[pl.ds(base, TOK_W)], i0_v)
        pltpu.sync_copy(pos1_hbm.at[pl.ds(base, TOK_W)], i1_v)

        def start(c, r):
            sl = pl.ds(c * CH2, CH2)
            pltpu.async_copy(y_hbm.at[i0_v.at[sl]], a_v[r], sem[r])
            pltpu.async_copy(y_hbm.at[i1_v.at[sl]], b_v[r], sem[r])
            pltpu.async_copy(sh_hbm.at[pl.ds(base + c * CH2, CH2)],
                             s_v[r], sem[r])

        def drain(r):
            pltpu.make_async_copy(y_hbm.at[i0_v.at[pl.ds(0, CH2)]],
                                  a_v[r], sem[r]).wait()
            pltpu.make_async_copy(y_hbm.at[i1_v.at[pl.ds(0, CH2)]],
                                  b_v[r], sem[r]).wait()
            pltpu.make_async_copy(sh_hbm.at[pl.ds(base, CH2)],
                                  s_v[r], sem[r]).wait()

        start(0, 0)
        for c in range(nch):
            r = c % 2
            if c + 1 < nch:
                start(c + 1, (c + 1) % 2)
            drain(r)

            def add_row(i, _):
                for k in range(D // NL):
                    sl = pl.ds(k * NL, NL)
                    a_v[r][i, sl] = (a_v[r][i, sl] + b_v[r][i, sl]
                                     + s_v[r][i, sl])
                return 0

            lax.fori_loop(0, CH2, add_row, 0)
            pltpu.sync_copy(a_v[r], out_hbm.at[pl.ds(base + c * CH2, CH2)])

    return dispatch_x, combine


def kernel(hidden_states, gate_w, Wg, Wu, Wd, Sg, Su, Sd, shared_gate_w):
    idx, w = _router(hidden_states, gate_w)

    flat_e = idx.reshape(-1)
    e_sorted, order = lax.sort(
        (flat_e, jnp.arange(NPAIR, dtype=jnp.int32)),
        dimension=0, is_stable=True, num_keys=1)
    ee = jnp.arange(E, dtype=jnp.int32)
    grp_start = jnp.searchsorted(e_sorted, ee, side="left").astype(jnp.int32)
    counts = (jnp.concatenate([grp_start[1:],
                               jnp.array([NPAIR], jnp.int32)]) - grp_start)
    nblk = (counts + BM - 1) // BM
    blk_end = jnp.cumsum(nblk)
    pad_off = ((blk_end - nblk) * BM).astype(jnp.int32)
    block_expert = jnp.minimum(
        jnp.searchsorted(blk_end, jnp.arange(NB, dtype=jnp.int32),
                         side="right").astype(jnp.int32), E - 1)
    delta_tbl = pad_off - grp_start
    delta = jnp.sum(jnp.where(e_sorted[:, None] == ee[None, :],
                              delta_tbl[None, :], 0), axis=1, dtype=jnp.int32)
    pp = jnp.arange(NPAIR, dtype=jnp.int32) + delta
    posf = jnp.zeros((NPAIR,), jnp.int32).at[order].set(pp)
    pos = posf.reshape(T, TOPK)

    dispatch_x, combine = _sc_kernels()
    x_pad, w_pad = dispatch_x(order, pp, w.reshape(-1), hidden_states)
    sh = _shared(hidden_states, Sg, Su, Sd, shared_gate_w)
    y = _grouped_mlp(block_expert, x_pad, Wg, Wu, Wd, w_pad[:, None])
    return combine(pos[:, 0], pos[:, 1], y, sh)


# speedup vs baseline: 1.1766x; 1.0037x over previous
import functools

import jax
import jax.numpy as jnp
from jax import lax
from jax.experimental import pallas as pl
from jax.experimental.pallas import tpu as pltpu
from jax.experimental.pallas import tpu_sc as plsc

T, D, E, DFF, DSH = 2048, 768, 64, 256, 512
TOPK = 2
NPAIR = T * TOPK
TB = 512
NTB = T // TB
BM = 128
NB = NPAIR // BM + E
NP = NB * BM

NWORK = 32
PAIR_W = NPAIR // NWORK
TOK_W = T // NWORK
CH2 = 16
NL = 16


def _router_body(x_ref, gate_w_ref, idx_ref, w_ref):
    x = x_ref[...]
    logits = jnp.dot(x, gate_w_ref[...], preferred_element_type=jnp.float32)
    mx = jnp.max(logits, axis=-1, keepdims=True)
    p = jnp.exp(logits - mx)
    probs = p / jnp.sum(p, axis=-1, keepdims=True)
    iota_e = jax.lax.broadcasted_iota(jnp.int32, (T, E), 1)
    m1 = jnp.max(probs, axis=-1, keepdims=True)
    i1 = jnp.min(jnp.where(probs == m1, iota_e, E), axis=-1)
    masked = jnp.where(iota_e == i1[:, None], -jnp.inf, probs)
    m2 = jnp.max(masked, axis=-1, keepdims=True)
    i2 = jnp.min(jnp.where(masked == m2, iota_e, E), axis=-1)
    denom = m1 + m2
    idx_ref[...] = jnp.concatenate([i1[:, None], i2[:, None]], axis=1)
    w_ref[...] = jnp.concatenate([m1 / denom, m2 / denom], axis=1)


def _router(x, gate_w):
    return pl.pallas_call(
        _router_body,
        out_shape=[
            jax.ShapeDtypeStruct((T, TOPK), jnp.int32),
            jax.ShapeDtypeStruct((T, TOPK), jnp.float32),
        ],
    )(x, gate_w)


def _shared_body(x_ref, Sg_ref, Su_ref, Sd_ref, sgw_ref, sh_ref):
    x = x_ref[...]
    xb = x.astype(jnp.bfloat16)
    g = jnp.dot(xb, Sg_ref[...].astype(jnp.bfloat16),
                preferred_element_type=jnp.float32)
    u = jnp.dot(xb, Su_ref[...].astype(jnp.bfloat16),
                preferred_element_type=jnp.float32)
    h = (g * jax.nn.sigmoid(g) * u).astype(jnp.bfloat16)
    sh = jnp.dot(h, Sd_ref[...].astype(jnp.bfloat16),
                 preferred_element_type=jnp.float32)
    sgate = jax.nn.sigmoid(jnp.dot(x, sgw_ref[...],
                                   preferred_element_type=jnp.float32))
    sh_ref[...] = sgate * sh


def _shared(x, Sg, Su, Sd, sgw):
    return pl.pallas_call(
        _shared_body,
        grid=(NTB,),
        in_specs=[
            pl.BlockSpec((TB, D), lambda t: (t, 0)),
            pl.BlockSpec((D, DSH), lambda t: (0, 0)),
            pl.BlockSpec((D, DSH), lambda t: (0, 0)),
            pl.BlockSpec((DSH, D), lambda t: (0, 0)),
            pl.BlockSpec((D, 1), lambda t: (0, 0)),
        ],
        out_specs=pl.BlockSpec((TB, D), lambda t: (t, 0)),
        out_shape=jax.ShapeDtypeStruct((T, D), jnp.float32),
        compiler_params=pltpu.CompilerParams(
            dimension_semantics=("arbitrary",)),
    )(x, Sg, Su, Sd, sgw)


def _grouped_mlp_body(be_ref, x_ref, Wg_ref, Wu_ref, Wd_ref, w_ref, y_ref,
                      wg_s, wu_s, wd_s):
    b = pl.program_id(0)
    prev = be_ref[jnp.maximum(b - 1, 0)]

    @pl.when((b == 0) | (be_ref[b] != prev))
    def _recast():
        wg_s[...] = Wg_ref[0].astype(jnp.bfloat16)
        wu_s[...] = Wu_ref[0].astype(jnp.bfloat16)
        wd_s[...] = Wd_ref[0].astype(jnp.bfloat16)

    xb = x_ref[...].astype(jnp.bfloat16)
    g = jnp.dot(xb, wg_s[...], preferred_element_type=jnp.float32)
    u = jnp.dot(xb, wu_s[...], preferred_element_type=jnp.float32)
    h = (g * jax.nn.sigmoid(g) * u).astype(jnp.bfloat16)
    eo = jnp.dot(h, wd_s[...], preferred_element_type=jnp.float32)
    y_ref[...] = w_ref[...] * eo


def _grouped_mlp(block_expert, x_pad, Wg, Wu, Wd, w_pad):
    grid_spec = pltpu.PrefetchScalarGridSpec(
        num_scalar_prefetch=1,
        grid=(NB,),
        in_specs=[
            pl.BlockSpec((BM, D), lambda b, be: (b, 0)),
            pl.BlockSpec((1, D, DFF), lambda b, be: (be[b], 0, 0)),
            pl.BlockSpec((1, D, DFF), lambda b, be: (be[b], 0, 0)),
            pl.BlockSpec((1, DFF, D), lambda b, be: (be[b], 0, 0)),
            pl.BlockSpec((BM, 1), lambda b, be: (b, 0)),
        ],
        out_specs=pl.BlockSpec((BM, D), lambda b, be: (b, 0)),
        scratch_shapes=[
            pltpu.VMEM((D, DFF), jnp.bfloat16),
            pltpu.VMEM((D, DFF), jnp.bfloat16),
            pltpu.VMEM((DFF, D), jnp.bfloat16),
        ],
    )
    return pl.pallas_call(
        _grouped_mlp_body,
        grid_spec=grid_spec,
        out_shape=jax.ShapeDtypeStruct((NP, D), jnp.float32),
        compiler_params=pltpu.CompilerParams(
            dimension_semantics=("arbitrary",)),
    )(block_expert, x_pad, Wg, Wu, Wd, w_pad)


@functools.lru_cache(maxsize=1)
def _sc_kernels():
    mesh = plsc.VectorSubcoreMesh(core_axis_name="c", subcore_axis_name="s",
                                  num_cores=2)

    @functools.partial(
        pl.kernel,
        mesh=mesh,
        out_type=[
            jax.ShapeDtypeStruct((NP, D), jnp.float32),
            jax.ShapeDtypeStruct((NP,), jnp.float32),
        ],
        scratch_types=[
            pltpu.VMEM((PAIR_W,), jnp.int32),
            pltpu.VMEM((PAIR_W,), jnp.int32),
            pltpu.VMEM((PAIR_W,), jnp.int32),
            pltpu.VMEM((PAIR_W,), jnp.float32),
            pltpu.VMEM((PAIR_W, D), jnp.float32),
            [pltpu.SemaphoreType.DMA] * 2,
        ],
    )
    def dispatch_x(order_hbm, pp_hbm, wflat_hbm, x_hbm, out_hbm, wpad_hbm,
                   ord_v, tok_v, pp_v, wv_v, rows_v, sem):
        wid = lax.axis_index("s") * 2 + lax.axis_index("c")
        base = wid * PAIR_W
        pltpu.sync_copy(order_hbm.at[pl.ds(base, PAIR_W)], ord_v)
        pltpu.sync_copy(pp_hbm.at[pl.ds(base, PAIR_W)], pp_v)
        for k in range(PAIR_W // NL):
            sl = pl.ds(k * NL, NL)
            tok_v[sl] = lax.shift_right_logical(ord_v[sl], 1)
        cp_w = pltpu.async_copy(wflat_hbm.at[ord_v], wv_v, sem[0])
        cp_x = pltpu.async_copy(x_hbm.at[tok_v], rows_v, sem[1])
        cp_w.wait()
        cp_x.wait()
        cp_w2 = pltpu.async_copy(wv_v, wpad_hbm.at[pp_v], sem[0])
        cp_x2 = pltpu.async_copy(rows_v, out_hbm.at[pp_v], sem[1])
        cp_w2.wait()
        cp_x2.wait()

    @functools.partial(
        pl.kernel,
        mesh=mesh,
        out_type=jax.ShapeDtypeStruct((T, D), jnp.float32),
        scratch_types=[
            pltpu.VMEM((TOK_W,), jnp.int32),
            pltpu.VMEM((TOK_W,), jnp.int32),
            [pltpu.VMEM((CH2, D), jnp.float32)] * 2,
            [pltpu.VMEM((CH2, D), jnp.float32)] * 2,
            [pltpu.VMEM((CH2, D), jnp.float32)] * 2,
            [[pltpu.SemaphoreType.DMA] * 3] * 2,
        ],
    )
    def combine(pos0_hbm, pos1_hbm, y_hbm, sh_hbm, out_hbm,
                i0_v, i1_v, a_v, b_v, s_v, sem):
        wid = lax.axis_index("s") * 2 + lax.axis_index("c")
        base = wid * TOK_W
        nch = TOK_W // CH2
        pltpu.sync_copy(pos0_hbm.at[pl.ds(base, TOK_W)], i0_v)
        pltpu.sync_copy(pos1_hbm.at[pl.ds(base, TOK_W)], i1_v)

        pend = [None, None]

        def start(c, r):
            sl = pl.ds(c * CH2, CH2)
            pend[r] = (
                pltpu.async_copy(y_hbm.at[i0_v.at[sl]], a_v[r], sem[r][0]),
                pltpu.async_copy(y_hbm.at[i1_v.at[sl]], b_v[r], sem[r][1]),
                pltpu.async_copy(sh_hbm.at[pl.ds(base + c * CH2, CH2)],
                                 s_v[r], sem[r][2]),
            )

        def drain(r):
            for cp in pend[r]:
                cp.wait()

        start(0, 0)
        for c in range(nch):
            r = c % 2
            if c + 1 < nch:
                start(c + 1, (c + 1) % 2)
            drain(r)

            def add_row(i, _):
                for k in range(D // NL):
                    sl = pl.ds(k * NL, NL)
                    a_v[r][i, sl] = (a_v[r][i, sl] + b_v[r][i, sl]
                                     + s_v[r][i, sl])
                return 0

            lax.fori_loop(0, CH2, add_row, 0)
            pltpu.sync_copy(a_v[r], out_hbm.at[pl.ds(base + c * CH2, CH2)])

    return dispatch_x, combine


def kernel(hidden_states, gate_w, Wg, Wu, Wd, Sg, Su, Sd, shared_gate_w):
    idx, w = _router(hidden_states, gate_w)

    flat_e = idx.reshape(-1)
    e_sorted, order = lax.sort(
        (flat_e, jnp.arange(NPAIR, dtype=jnp.int32)),
        dimension=0, is_stable=True, num_keys=1)
    ee = jnp.arange(E, dtype=jnp.int32)
    grp_start = jnp.searchsorted(e_sorted, ee, side="left").astype(jnp.int32)
    counts = (jnp.concatenate([grp_start[1:],
                               jnp.array([NPAIR], jnp.int32)]) - grp_start)
    nblk = (counts + BM - 1) // BM
    blk_end = jnp.cumsum(nblk)
    pad_off = ((blk_end - nblk) * BM).astype(jnp.int32)
    block_expert = jnp.minimum(
        jnp.searchsorted(blk_end, jnp.arange(NB, dtype=jnp.int32),
                         side="right").astype(jnp.int32), E - 1)
    delta_tbl = pad_off - grp_start
    delta = jnp.sum(jnp.where(e_sorted[:, None] == ee[None, :],
                              delta_tbl[None, :], 0), axis=1, dtype=jnp.int32)
    pp = jnp.arange(NPAIR, dtype=jnp.int32) + delta
    posf = jnp.zeros((NPAIR,), jnp.int32).at[order].set(pp)
    pos = posf.reshape(T, TOPK)

    dispatch_x, combine = _sc_kernels()
    x_pad, w_pad = dispatch_x(order, pp, w.reshape(-1), hidden_states)
    sh = _shared(hidden_states, Sg, Su, Sd, shared_gate_w)
    y = _grouped_mlp(block_expert, x_pad, Wg, Wu, Wd, w_pad[:, None])
    return combine(pos[:, 0], pos[:, 1], y, sh)
